# baseline (speedup 1.0000x reference)
import functools

import jax
import jax.numpy as jnp
from jax import lax
from jax.experimental import pallas as pl
from jax.experimental.pallas import tpu as pltpu
from jax.experimental.pallas import tpu_sc as plsc

_BLK = 100
_K = 8
_NBUF = 2


@functools.cache
def _make_gather(nb, nl, num_emb, d):
    info = plsc.get_sparse_core_info()
    nc, ns = info.num_cores, info.num_subcores
    nw = nc * ns
    n_total = nb * nl
    sent_per_group = _K * _BLK // nl
    blocks_total = n_total // _BLK
    blocks_w = blocks_total // nw
    groups = blocks_w // _K
    assert groups % _NBUF == 0 and (_K * _BLK) % nl == 0
    stripe = -(-num_emb // ns)
    mesh = plsc.VectorSubcoreMesh(core_axis_name="c", subcore_axis_name="s")

    @functools.partial(
        pl.kernel,
        mesh=mesh,
        out_type=jax.ShapeDtypeStruct((nb, nl, d), jnp.float32),
        scratch_types=[
            pltpu.VMEM_SHARED((num_emb, d), jnp.float32),
            pltpu.VMEM((_K, _BLK), jnp.int32),
            pltpu.VMEM((_K, _BLK), jnp.int32),
            pltpu.VMEM((sent_per_group, nl, d), jnp.float32),
            pltpu.VMEM((sent_per_group, nl, d), jnp.float32),
            pltpu.SemaphoreType.DMA,
            pltpu.SemaphoreType.DMA,
            pltpu.SemaphoreType.DMA,
            pltpu.SemaphoreType.DMA,
            pltpu.SemaphoreType.DMA,
        ],
        compiler_params=pltpu.CompilerParams(use_tc_tiling_on_sc=False),
    )
    def gather_kernel(idx_hbm, table_hbm, out_hbm, table_sh, idx0, idx1,
                      rows0, rows1, sem_i0, sem_i1, sem_g, sem_o0, sem_o1):
        idx_v = (idx0, idx1)
        rows_v = (rows0, rows1)
        sem_i = (sem_i0, sem_i1)
        sem_o = (sem_o0, sem_o1)
        sid = lax.axis_index("s")
        wid = sid * nc + lax.axis_index("c")
        blk0 = wid * blocks_w
        snt0 = wid * (blocks_w * _BLK // nl)

        start = jnp.minimum(sid * stripe, num_emb - stripe)
        pltpu.sync_copy(
            table_hbm.at[pl.ds(start, stripe)], table_sh.at[pl.ds(start, stripe)]
        )
        plsc.subcore_barrier()

        def idx_src(g):
            return idx_hbm.at[pl.ds(blk0 + g * _K, _K)]

        def do_group(g, b, wait_out):
            if wait_out:
                pltpu.make_async_copy(
                    rows_v[b], out_hbm.at[pl.ds(0, sent_per_group)], sem_o[b]
                ).wait()
            pltpu.make_async_copy(idx_src(0), idx_v[b], sem_i[b]).wait()
            copies = [
                pltpu.async_copy(
                    table_sh.at[idx_v[b].at[j]],
                    rows_v[b].at[j * _BLK // nl, pl.ds(j * _BLK % nl, _BLK)],
                    sem_g,
                )
                for j in range(_K)
            ]
            for c in copies:
                c.wait()
            gn = jnp.minimum(g + _NBUF, groups - 1)
            pltpu.async_copy(idx_src(gn), idx_v[b], sem_i[b])
            pltpu.async_copy(
                rows_v[b],
                out_hbm.at[pl.ds(snt0 + g * sent_per_group, sent_per_group)],
                sem_o[b],
            )

        for b in range(_NBUF):
            pltpu.async_copy(idx_src(b), idx_v[b], sem_i[b])
        for b in range(_NBUF):
            do_group(jnp.int32(b), b, wait_out=False)

        def pair(p, carry):
            for b in range(_NBUF):
                do_group(p * _NBUF + b, b, wait_out=True)
            return carry

        lax.fori_loop(1, groups // _NBUF, pair, 0)

        for b in range(_NBUF):
            pltpu.make_async_copy(idx_src(0), idx_v[b], sem_i[b]).wait()
            pltpu.make_async_copy(
                rows_v[b], out_hbm.at[pl.ds(0, sent_per_group)], sem_o[b]
            ).wait()

    return gather_kernel


def kernel(sent_list, table):
    nb, nl = sent_list.shape
    d = table.shape[1]
    idx2d = sent_list.reshape(nb * nl // _BLK, _BLK).astype(jnp.int32)
    return _make_gather(nb, nl, table.shape[0], d)(
        idx2d, table.astype(jnp.float32)
    )
